# Initial kernel scaffold; baseline (speedup 1.0000x reference)
import jax
import jax.numpy as jnp
from jax.experimental import pallas as pl


def kernel(seqs, seq_lens, temporal_mask_embed):
    raise NotImplementedError("write your pallas kernel here")


# speedup vs baseline: 1.4013x; 1.4013x over previous
import jax
import jax.numpy as jnp
from jax.experimental import pallas as pl

_SPAN_LEN = 10
_MAX_MASK_PROB = 0.65
_MASK_KEY_SEED = 1234


def _span_starts(seq_lens, batch, seq_len):
    rate = _MAX_MASK_PROB / _SPAN_LEN
    num_spans_per_row = rate * (seq_lens.astype(jnp.float32) - 1.0)
    num_spans = jnp.min(num_spans_per_row.astype(jnp.int32))
    ns_min = int(rate * ((seq_len // 2) - 1.0))
    ns_max = int(rate * (seq_len - 1.0))
    ns_lo = max(ns_min - 1, 2)
    ns_hi = ns_max + 1
    ns_pad = ns_hi
    key = jax.random.key(_MASK_KEY_SEED)

    def make_branch(ns):
        def branch(operands):
            branch_key, row_lens = operands
            span_start_range = row_lens - _SPAN_LEN + 1
            span_start_range = jnp.repeat(span_start_range, ns)
            u = jax.random.uniform(branch_key, (batch * ns,), dtype=jnp.float32)
            offs = (span_start_range.astype(jnp.float32) * u).astype(row_lens.dtype)
            offs = offs.reshape(batch, ns)
            if ns < ns_pad:
                pad = jnp.full((batch, ns_pad - ns), -(_SPAN_LEN + 1), jnp.int32)
                offs = jnp.concatenate([offs, pad], axis=1)
            return offs
        return branch

    branches = [make_branch(ns) for ns in range(ns_lo, ns_hi + 1)]
    return jax.lax.switch(num_spans - ns_lo, branches, (key, seq_lens))


def _fill_body(starts_ref, embed_ref, seq_ref, out_ref, mask_ref):
    seq_blk = seq_ref[0]
    starts = starts_ref[0]
    s = seq_blk.shape[0]
    j = pl.program_id(1)
    pos = jax.lax.broadcasted_iota(jnp.int32, (s, 1), 0) + j * s
    d = pos - starts
    hit = (d >= 0) & (d < _SPAN_LEN)
    mask = jnp.any(hit, axis=1, keepdims=True)
    out_ref[0] = jnp.where(mask, embed_ref[0], seq_blk)
    mask_ref[0] = mask


def kernel(seqs, seq_lens, temporal_mask_embed):
    batch, seq_len, model_dim = seqs.shape
    starts = _span_starts(seq_lens, batch, seq_len)
    ns_pad = starts.shape[1]
    starts3 = starts.reshape(batch, 1, ns_pad)
    embed2 = temporal_mask_embed.reshape(1, model_dim)

    s_blk = 512
    grid = (batch, seq_len // s_blk)
    masked, mask3 = pl.pallas_call(
        _fill_body,
        grid=grid,
        in_specs=[
            pl.BlockSpec((1, 1, ns_pad), lambda i, j: (i, 0, 0)),
            pl.BlockSpec((1, model_dim), lambda i, j: (0, 0)),
            pl.BlockSpec((1, s_blk, model_dim), lambda i, j: (i, j, 0)),
        ],
        out_specs=[
            pl.BlockSpec((1, s_blk, model_dim), lambda i, j: (i, j, 0)),
            pl.BlockSpec((1, s_blk, 1), lambda i, j: (i, j, 0)),
        ],
        out_shape=[
            jax.ShapeDtypeStruct((batch, seq_len, model_dim), seqs.dtype),
            jax.ShapeDtypeStruct((batch, seq_len, 1), jnp.bool_),
        ],
    )(starts3, embed2, seqs)
    return masked, mask3.reshape(batch, seq_len)
